# baseline (speedup 1.0000x reference)
import functools

import jax
import jax.numpy as jnp
from jax import lax
from jax.experimental import pallas as pl
from jax.experimental.pallas import tpu as pltpu
from jax.experimental.pallas import tpu_sc as plsc

N_ATOMS = 131072
D = 256
N_CRYSTALS = 1024
ATOMS_PER = 128

NC, NS = 2, 16
NW = NC * NS
C_PER_W = N_CRYSTALS // NW
LANES = 16
N_CHUNK = D // LANES
HALF = ATOMS_PER // 2
H_PER_W = C_PER_W * 2
N_BUF = 4
R_STEP = 8


def _sc_segment_sum_body(idx_hbm, table_hbm, out_hbm,
                         idx_v, rows_a, rows_b, rows_c, rows_d, out_v,
                         idxb_a, idxb_b, idxb_c, idxb_d,
                         sem_a, sem_b, sem_c, sem_d):
    wid = lax.axis_index("s") * NC + lax.axis_index("c")
    base = wid * C_PER_W

    pltpu.sync_copy(idx_hbm.at[pl.ds(base * 2, H_PER_W)], idx_v)

    bufs = (rows_a, rows_b, rows_c, rows_d)
    idxbs = (idxb_a, idxb_b, idxb_c, idxb_d)
    sems = (sem_a, sem_b, sem_c, sem_d)

    def gather(h, b):
        for c in range(HALF // LANES):
            idxbs[b][pl.ds(c * LANES, LANES)] = idx_v[h, pl.ds(c * LANES,
                                                               LANES)]
        pltpu.async_copy(table_hbm.at[idxbs[b]], bufs[b], sems[b])

    def wait(b):
        pltpu.make_async_copy(table_hbm.at[idxbs[b]], bufs[b],
                              sems[b]).wait()

    def reduce_half(b, acc0):
        buf = bufs[b]

        @plsc.parallel_loop(0, HALF, step=R_STEP, unroll=4, carry=acc0)
        def acc(r, a):
            for u in range(R_STEP):
                a = tuple(a[c] + buf[r + u, pl.ds(c * LANES, LANES)]
                          for c in range(N_CHUNK))
            return a

        return acc

    for b in range(N_BUF):
        gather(b, b)

    @pl.loop(0, H_PER_W, step=N_BUF)
    def _(h0):
        j0 = h0 // 2
        for cb in range(N_BUF // 2):
            acc = tuple(jnp.zeros((LANES,), jnp.float32)
                        for _ in range(N_CHUNK))
            for hb in range(2):
                b = cb * 2 + hb
                h = h0 + b
                wait(b)
                acc = reduce_half(b, acc)

                @pl.when(h + N_BUF < H_PER_W)
                def _():
                    gather(h + N_BUF, b)

            for c in range(N_CHUNK):
                out_v[j0 + cb, pl.ds(c * LANES, LANES)] = acc[c]

    pltpu.sync_copy(out_v, out_hbm.at[pl.ds(base, C_PER_W)])


_sc_segment_sum = functools.partial(
    pl.kernel,
    out_type=jax.ShapeDtypeStruct((N_CRYSTALS, D), jnp.float32),
    mesh=plsc.VectorSubcoreMesh(core_axis_name="c", subcore_axis_name="s"),
    scratch_types=[
        pltpu.VMEM((H_PER_W, HALF), jnp.int32),
        pltpu.VMEM((HALF, D), jnp.float32),
        pltpu.VMEM((HALF, D), jnp.float32),
        pltpu.VMEM((HALF, D), jnp.float32),
        pltpu.VMEM((HALF, D), jnp.float32),
        pltpu.VMEM((C_PER_W, D), jnp.float32),
        pltpu.VMEM((HALF,), jnp.int32),
        pltpu.VMEM((HALF,), jnp.int32),
        pltpu.VMEM((HALF,), jnp.int32),
        pltpu.VMEM((HALF,), jnp.int32),
        pltpu.SemaphoreType.DMA,
        pltpu.SemaphoreType.DMA,
        pltpu.SemaphoreType.DMA,
        pltpu.SemaphoreType.DMA,
    ],
)(_sc_segment_sum_body)


def _tc_gate_body(x_ref, w_ref, b_ref, o_ref):
    x = x_ref[...] * jnp.float32(1.0 / ATOMS_PER)
    z = lax.dot_general(x, w_ref[...], (((1,), (1,)), ((), ())),
                        preferred_element_type=jnp.float32)
    o_ref[...] = jax.nn.sigmoid(z + b_ref[...])


def _tc_gate(sums, W, b):
    return pl.pallas_call(
        _tc_gate_body,
        out_shape=jax.ShapeDtypeStruct((N_CRYSTALS, D), jnp.float32),
    )(sums, W, b.reshape(1, D))


@jax.jit
def kernel(crystal_atom_fea, crystal_atom_idx, W, b):
    idx2 = crystal_atom_idx.reshape(N_CRYSTALS * 2, HALF)
    sums = _sc_segment_sum(idx2, crystal_atom_fea)
    return _tc_gate(sums, W, b)


# speedup vs baseline: 1.0176x; 1.0176x over previous
import functools

import jax
import jax.numpy as jnp
from jax import lax
from jax.experimental import pallas as pl
from jax.experimental.pallas import tpu as pltpu
from jax.experimental.pallas import tpu_sc as plsc

N_ATOMS = 131072
D = 256
N_CRYSTALS = 1024
ATOMS_PER = 128

NC, NS = 2, 16
NW = NC * NS
C_PER_W = N_CRYSTALS // NW
LANES = 16
N_CHUNK = D // LANES
HALF = ATOMS_PER // 2
H_PER_W = C_PER_W * 2
N_BUF = 4
R_STEP = 8


def _sc_segment_sum_body(idx_hbm, table_hbm, out_hbm,
                         idx_v, rows_a, rows_b, rows_c, rows_d, out_v,
                         sem_a, sem_b, sem_c, sem_d):
    wid = lax.axis_index("s") * NC + lax.axis_index("c")
    base = wid * C_PER_W

    pltpu.sync_copy(idx_hbm.at[pl.ds(base * 2, H_PER_W)], idx_v)

    bufs = (rows_a, rows_b, rows_c, rows_d)
    sems = (sem_a, sem_b, sem_c, sem_d)

    def gather(h, b):
        pltpu.async_copy(table_hbm.at[idx_v.at[h]], bufs[b], sems[b])

    def wait(b):
        pltpu.make_async_copy(table_hbm.at[idx_v.at[0]], bufs[b],
                              sems[b]).wait()

    def reduce_half(b, acc0):
        buf = bufs[b]

        @plsc.parallel_loop(0, HALF, step=R_STEP, unroll=4, carry=acc0)
        def acc(r, a):
            for u in range(R_STEP):
                a = tuple(a[c] + buf[r + u, pl.ds(c * LANES, LANES)]
                          for c in range(N_CHUNK))
            return a

        return acc

    for b in range(N_BUF):
        gather(b, b)

    @pl.loop(0, H_PER_W, step=N_BUF)
    def _(h0):
        j0 = h0 // 2
        for cb in range(N_BUF // 2):
            acc = tuple(jnp.zeros((LANES,), jnp.float32)
                        for _ in range(N_CHUNK))
            for hb in range(2):
                b = cb * 2 + hb
                h = h0 + b
                wait(b)
                acc = reduce_half(b, acc)

                @pl.when(h + N_BUF < H_PER_W)
                def _():
                    gather(h + N_BUF, b)

            for c in range(N_CHUNK):
                out_v[j0 + cb, pl.ds(c * LANES, LANES)] = acc[c]

    pltpu.sync_copy(out_v, out_hbm.at[pl.ds(base, C_PER_W)])


_sc_segment_sum = functools.partial(
    pl.kernel,
    out_type=jax.ShapeDtypeStruct((N_CRYSTALS, D), jnp.float32),
    mesh=plsc.VectorSubcoreMesh(core_axis_name="c", subcore_axis_name="s"),
    scratch_types=[
        pltpu.VMEM((H_PER_W, HALF), jnp.int32),
        pltpu.VMEM((HALF, D), jnp.float32),
        pltpu.VMEM((HALF, D), jnp.float32),
        pltpu.VMEM((HALF, D), jnp.float32),
        pltpu.VMEM((HALF, D), jnp.float32),
        pltpu.VMEM((C_PER_W, D), jnp.float32),
        pltpu.SemaphoreType.DMA,
        pltpu.SemaphoreType.DMA,
        pltpu.SemaphoreType.DMA,
        pltpu.SemaphoreType.DMA,
    ],
)(_sc_segment_sum_body)


def _tc_gate_body(x_ref, w_ref, b_ref, o_ref):
    x = x_ref[...] * jnp.float32(1.0 / ATOMS_PER)
    z = lax.dot_general(x, w_ref[...], (((1,), (1,)), ((), ())),
                        preferred_element_type=jnp.float32)
    o_ref[...] = jax.nn.sigmoid(z + b_ref[...])


def _tc_gate(sums, W, b):
    return pl.pallas_call(
        _tc_gate_body,
        out_shape=jax.ShapeDtypeStruct((N_CRYSTALS, D), jnp.float32),
    )(sums, W, b.reshape(1, D))


@jax.jit
def kernel(crystal_atom_fea, crystal_atom_idx, W, b):
    idx2 = crystal_atom_idx.reshape(N_CRYSTALS * 2, HALF)
    sums = _sc_segment_sum(idx2, crystal_atom_fea)
    return _tc_gate(sums, W, b)
